# Initial kernel scaffold; baseline (speedup 1.0000x reference)
import jax
import jax.numpy as jnp
from jax.experimental import pallas as pl


def kernel(x, edge_index, rbf, distances, k_idx, j_idx, i_idx, angles, cbf, batch, We1_0, We2_0, Wn_0, bn_0, We1_1, We2_1, Wn_1, bn_1, We1_2, We2_2, Wn_2, bn_2, Wo1, bo1, Wo2, bo2):
    raise NotImplementedError("write your pallas kernel here")


# speedup vs baseline: 17.7052x; 17.7052x over previous
import jax
import jax.numpy as jnp
from jax import lax
from jax.experimental import pallas as pl
from jax.experimental.pallas import tpu as pltpu
from jax.experimental.pallas import tpu_sc as plsc

_N = 10000
_E = 320000
_T = 640000
_H = 64
_NG = 32
_NC = 2
_NS = 16
_NW = _NC * _NS
_NPADT = _N + 16
_RPS = 640
_NSH = _RPS * _NS

_TPW = _T // _NW
_CH = 2000
_NCHK = _TPW // _CH
_S = 64

_EPW = _E // _NW
_EB = 400
_NEB = _EPW // _EB

_sc_mesh = plsc.VectorSubcoreMesh(
    core_axis_name="c", subcore_axis_name="s",
    num_cores=_NC, num_subcores=_NS)


def _zero_stripe(zbuf, sh, s):
    def _zrow(r, carry):
        for m in range(4):
            zbuf[r, pl.ds(m * 16, 16)] = jnp.zeros((16,), jnp.float32)
        return carry
    lax.fori_loop(0, 64, _zrow, 0)

    def _zcp(i, carry):
        pltpu.sync_copy(zbuf, sh.at[pl.ds(s * _RPS + i * 64, 64)])
        return carry
    lax.fori_loop(0, _RPS // 64, _zcp, 0)


def _trip_body(kidx, jidx, cbf, xw, rbfw, wc, out,
               kbuf, jbuf, cbfbuf, ksurv, jsurv, tsurv,
               kb, jb, tb, xwb, rbfb, yb, wcb, zbuf, aggsh, sem):
    c = lax.axis_index("c")
    s = lax.axis_index("s")
    wid = c * _NS + s

    _zero_stripe(zbuf, aggsh, s)

    def _sinit(v, carry):
        z = jnp.zeros((16,), jnp.int32)
        ksurv[pl.ds(v * 16, 16)] = z
        tsurv[pl.ds(v * 16, 16)] = z
        return carry
    lax.fori_loop(0, _CH // 16, _sinit, 0)

    pltpu.sync_copy(wc, wcb)
    plsc.subcore_barrier()

    wcv = [[wcb[r, pl.ds(m * 16, 16)] for m in range(4)] for r in range(6)]
    base0 = wid * _TPW

    def _chunk(ci, carry):
        cb = base0 + ci * _CH
        pltpu.sync_copy(kidx.at[pl.ds(cb, _CH)], kbuf)
        pltpu.sync_copy(jidx.at[pl.ds(cb, _CH)], jbuf)
        pltpu.sync_copy(cbf.at[pl.ds(cb, _CH)], cbfbuf)

        def _jinit(v, cr):
            jsurv[pl.ds(v * 16, 16)] = jnp.full((16,), _N, jnp.int32)
            return cr
        lax.fori_loop(0, _CH // 16, _jinit, 0)

        def _cvec(v, w):
            jv = jbuf[pl.ds(v * 16, 16)]
            kv = kbuf[pl.ds(v * 16, 16)]
            m = jv < _N
            mi = m.astype(jnp.int32)
            pos = w + plsc.cumsum(mi) - 1
            plsc.store_scatter(jsurv, [pos], jv, mask=m)
            plsc.store_scatter(ksurv, [pos], kv, mask=m)
            tv = lax.iota(jnp.int32, 16) + v * 16
            plsc.store_scatter(tsurv, [pos], tv, mask=m)
            return w + jnp.sum(mi)
        w = lax.fori_loop(0, _CH // 16, _cvec, jnp.int32(0))

        nb = (w + (_S - 1)) // _S

        def _batch(b, cr):
            off = b * _S
            for v in range(_S // 16):
                kb[pl.ds(v * 16, 16)] = ksurv[pl.ds(off + v * 16, 16)]
                jb[pl.ds(v * 16, 16)] = jsurv[pl.ds(off + v * 16, 16)]
                tb[pl.ds(v * 16, 16)] = tsurv[pl.ds(off + v * 16, 16)]
            cp1 = pltpu.async_copy(xw.at[kb], xwb, sem)
            cp2 = pltpu.async_copy(rbfw.at[jb], rbfb, sem)
            cp1.wait()
            cp2.wait()

            def _grp(g, rc):
                tvec = tb[pl.ds(g * 16, 16)]
                cvals = [plsc.load_gather(
                    cbfbuf, [tvec, jnp.full((16,), r, jnp.int32)])
                    for r in range(6)]
                for ii in range(16):
                    i = g * 16 + ii
                    for m in range(4):
                        acc = (xwb[i, pl.ds(m * 16, 16)]
                               + rbfb[i, pl.ds(m * 16, 16)])
                        for r in range(6):
                            acc = acc + cvals[r][ii] * wcv[r][m]
                        yb[i, pl.ds(m * 16, 16)] = jnp.maximum(acc, 0.0)
                return rc
            lax.fori_loop(0, _S // 16, _grp, 0)
            pltpu.sync_copy(yb, aggsh.at[jb], add=True)
            return cr
        lax.fori_loop(0, nb, _batch, 0)
        return carry
    lax.fori_loop(0, _NCHK, _chunk, 0)

    plsc.subcore_barrier()
    pltpu.sync_copy(aggsh.at[pl.ds(s * _RPS, _RPS)],
                    out.at[c, pl.ds(s * _RPS, _RPS)])


_SC_PARAMS = pltpu.CompilerParams(use_tc_tiling_on_sc=False,
                                  needs_layout_passes=False)

_trip_call = pl.kernel(
    _trip_body,
    out_type=jax.ShapeDtypeStruct((_NC, _NSH, _H), jnp.float32),
    mesh=_sc_mesh,
    compiler_params=_SC_PARAMS,
    scratch_types=[
        pltpu.VMEM((_CH,), jnp.int32),
        pltpu.VMEM((_CH,), jnp.int32),
        pltpu.VMEM((_CH, 6), jnp.float32),
        pltpu.VMEM((_CH,), jnp.int32),
        pltpu.VMEM((_CH,), jnp.int32),
        pltpu.VMEM((_CH,), jnp.int32),
        pltpu.VMEM((_S,), jnp.int32),
        pltpu.VMEM((_S,), jnp.int32),
        pltpu.VMEM((_S,), jnp.int32),
        pltpu.VMEM((_S, _H), jnp.float32),
        pltpu.VMEM((_S, _H), jnp.float32),
        pltpu.VMEM((_S, _H), jnp.float32),
        pltpu.VMEM((6, _H), jnp.float32),
        pltpu.VMEM((64, _H), jnp.float32),
        pltpu.VMEM_SHARED((_NSH, _H), jnp.float32),
        pltpu.SemaphoreType.DMA,
    ],
)


def _edge_body(srci, dsti, xs, aggw, out,
               srcb, dstb, xsb, awb, yb, zbuf, nodesh, sem):
    c = lax.axis_index("c")
    s = lax.axis_index("s")
    wid = c * _NS + s

    _zero_stripe(zbuf, nodesh, s)
    plsc.subcore_barrier()

    base0 = wid * _EPW

    def _b(b, carry):
        bb = base0 + b * _EB
        pltpu.sync_copy(srci.at[pl.ds(bb, _EB)], srcb)
        pltpu.sync_copy(dsti.at[pl.ds(bb, _EB)], dstb)
        cp1 = pltpu.async_copy(xs.at[srcb], xsb, sem)
        cp2 = pltpu.async_copy(aggw.at[dstb], awb, sem)
        cp1.wait()
        cp2.wait()

        def _row(i, rc):
            for m in range(4):
                yb[i, pl.ds(m * 16, 16)] = jnp.maximum(
                    xsb[i, pl.ds(m * 16, 16)] + awb[i, pl.ds(m * 16, 16)], 0.0)
            return rc
        lax.fori_loop(0, _EB, _row, 0)
        pltpu.sync_copy(yb, nodesh.at[dstb], add=True)
        return carry
    lax.fori_loop(0, _NEB, _b, 0)

    plsc.subcore_barrier()
    pltpu.sync_copy(nodesh.at[pl.ds(s * _RPS, _RPS)],
                    out.at[c, pl.ds(s * _RPS, _RPS)])


_edge_call = pl.kernel(
    _edge_body,
    out_type=jax.ShapeDtypeStruct((_NC, _NSH, _H), jnp.float32),
    mesh=_sc_mesh,
    compiler_params=_SC_PARAMS,
    scratch_types=[
        pltpu.VMEM((_EB,), jnp.int32),
        pltpu.VMEM((_EB,), jnp.int32),
        pltpu.VMEM((_EB, _H), jnp.float32),
        pltpu.VMEM((_EB, _H), jnp.float32),
        pltpu.VMEM((_EB, _H), jnp.float32),
        pltpu.VMEM((64, _H), jnp.float32),
        pltpu.VMEM_SHARED((_NSH, _H), jnp.float32),
        pltpu.SemaphoreType.DMA,
    ],
)


_RB = 2000


def _rbfw_body(r_ref, w0_ref, w1_ref, w2_ref, o0_ref, o1_ref, o2_ref):
    r = r_ref[...]
    o0_ref[...] = jnp.dot(r, w0_ref[...], preferred_element_type=jnp.float32)
    o1_ref[...] = jnp.dot(r, w1_ref[...], preferred_element_type=jnp.float32)
    o2_ref[...] = jnp.dot(r, w2_ref[...], preferred_element_type=jnp.float32)


def _rbfw_tables(rbf_n, w0, w1, w2):
    return pl.pallas_call(
        _rbfw_body,
        grid=(_N // _RB,),
        in_specs=[pl.BlockSpec((_RB, 8), lambda i: (i, 0))]
        + [pl.BlockSpec((8, _H), lambda i: (0, 0))] * 3,
        out_specs=[pl.BlockSpec((_RB, _H), lambda i: (i, 0))] * 3,
        out_shape=[jax.ShapeDtypeStruct((_N, _H), jnp.float32)] * 3,
    )(rbf_n, w0, w1, w2)


def _pre_body(h_ref, w1_ref, w2_ref, xw_ref, xs_ref):
    h = h_ref[...]
    xw_ref[...] = jnp.dot(h, w1_ref[...], preferred_element_type=jnp.float32)
    xs_ref[...] = jnp.dot(h, w2_ref[...], preferred_element_type=jnp.float32)


def _pre(h, w1x, w2x):
    in_c = h.shape[1]
    return pl.pallas_call(
        _pre_body,
        grid=(_N // _RB,),
        in_specs=[pl.BlockSpec((_RB, in_c), lambda i: (i, 0)),
                  pl.BlockSpec((in_c, _H), lambda i: (0, 0)),
                  pl.BlockSpec((in_c, _H), lambda i: (0, 0))],
        out_specs=[pl.BlockSpec((_RB, _H), lambda i: (i, 0))] * 2,
        out_shape=[jax.ShapeDtypeStruct((_N, _H), jnp.float32)] * 2,
    )(h, w1x, w2x)


def _aggw_body(agg_ref, w_ref, o_ref):
    a = agg_ref[0] + agg_ref[1]
    o_ref[...] = jnp.dot(a, w_ref[...], preferred_element_type=jnp.float32)


def _aggw(agg2, w):
    return pl.pallas_call(
        _aggw_body,
        grid=(_N // _RB,),
        in_specs=[pl.BlockSpec((_NC, _RB, _H), lambda i: (0, i, 0)),
                  pl.BlockSpec((_H, _H), lambda i: (0, 0))],
        out_specs=pl.BlockSpec((_RB, _H), lambda i: (i, 0)),
        out_shape=jax.ShapeDtypeStruct((_N, _H), jnp.float32),
    )(agg2, w)


def _hnext_body(h_ref, nd_ref, wx_ref, wa_ref, bn_ref, o_ref):
    nd = nd_ref[0] + nd_ref[1]
    o_ref[...] = (jnp.dot(h_ref[...], wx_ref[...], preferred_element_type=jnp.float32)
                  + jnp.dot(nd, wa_ref[...], preferred_element_type=jnp.float32)
                  + bn_ref[...][0:1])


def _hnext(h, node2, wx, wa, bn8):
    in_c = h.shape[1]
    return pl.pallas_call(
        _hnext_body,
        grid=(_N // _RB,),
        in_specs=[pl.BlockSpec((_RB, in_c), lambda i: (i, 0)),
                  pl.BlockSpec((_NC, _RB, _H), lambda i: (0, i, 0)),
                  pl.BlockSpec((in_c, _H), lambda i: (0, 0)),
                  pl.BlockSpec((_H, _H), lambda i: (0, 0)),
                  pl.BlockSpec((8, _H), lambda i: (0, 0))],
        out_specs=pl.BlockSpec((_RB, _H), lambda i: (i, 0)),
        out_shape=jax.ShapeDtypeStruct((_N, _H), jnp.float32),
    )(h, node2, wx, wa, bn8)


def _final_body(h_ref, b_ref, wo1_ref, bo1_ref, wo2_ref, bo2_ref, o_ref):
    bt = b_ref[...][0:1]
    gids = lax.broadcasted_iota(jnp.int32, (_NG, _N), 0)
    oh = (jnp.broadcast_to(bt, (_NG, _N)) == gids).astype(jnp.float32)
    sums = lax.dot_general(oh, h_ref[...], (((1,), (0,)), ((), ())),
                           preferred_element_type=jnp.float32)
    counts = jnp.sum(oh, axis=1, keepdims=True)
    pooled = jnp.maximum(sums / jnp.maximum(counts, 1.0), 0.0)
    hid = jnp.maximum(
        jnp.dot(pooled, wo1_ref[...], preferred_element_type=jnp.float32)
        + bo1_ref[...][0:1], 0.0)
    o_ref[...] = (jnp.dot(hid, wo2_ref[...], preferred_element_type=jnp.float32)
                  + bo2_ref[...][0:1])


def _final(h, batch8, wo1, bo18, wo2p, bo28):
    return pl.pallas_call(
        _final_body,
        in_specs=[pl.BlockSpec((_N, _H), lambda: (0, 0)),
                  pl.BlockSpec((8, _N), lambda: (0, 0)),
                  pl.BlockSpec((_H, _H), lambda: (0, 0)),
                  pl.BlockSpec((8, _H), lambda: (0, 0)),
                  pl.BlockSpec((_H, 128), lambda: (0, 0)),
                  pl.BlockSpec((8, 128), lambda: (0, 0))],
        out_specs=pl.BlockSpec((_NG, 128), lambda: (0, 0)),
        out_shape=jax.ShapeDtypeStruct((_NG, 128), jnp.float32),
    )(h, batch8, wo1, bo18, wo2p, bo28)


def kernel(x, edge_index, rbf, distances, k_idx, j_idx, i_idx, angles, cbf,
           batch, We1_0, We2_0, Wn_0, bn_0, We1_1, We2_1, Wn_1, bn_1,
           We1_2, We2_2, Wn_2, bn_2, Wo1, bo1, Wo2, bo2):
    src = edge_index[0]
    dst = edge_index[1]
    params = [(We1_0, We2_0, Wn_0, bn_0),
              (We1_1, We2_1, Wn_1, bn_1),
              (We1_2, We2_2, Wn_2, bn_2)]

    rbf_n = jnp.pad(rbf[:_N], ((0, 0), (0, 2)))
    in_cs = [128, 64, 64]
    wr = [jnp.pad(params[l][0][in_cs[l]:in_cs[l] + 6], ((0, 2), (0, 0)))
          for l in range(3)]
    rbfw = _rbfw_tables(rbf_n, wr[0], wr[1], wr[2])
    zpad = jnp.zeros((_NPADT - _N, _H), jnp.float32)
    rbfw = [jnp.concatenate([t, zpad], axis=0) for t in rbfw]

    h = x
    for l in range(3):
        We1, We2, Wn, bn = params[l]
        in_c = in_cs[l]
        xw, xs = _pre(h, We1[:in_c], We2[:in_c])
        agg2 = _trip_call(k_idx, j_idx, cbf, xw, rbfw[l], We1[in_c + 6:])
        aggw = _aggw(agg2, We2[in_c:])
        node2 = _edge_call(src, dst, xs, aggw)
        h = _hnext(h, node2, Wn[:in_c], Wn[in_c:],
                   jnp.broadcast_to(bn[None, :], (8, _H)))

    batch8 = jnp.broadcast_to(batch[None, :], (8, _N)).astype(jnp.int32)
    wo2p = jnp.pad(Wo2, ((0, 0), (0, 127)))
    bo28 = jnp.broadcast_to(jnp.pad(bo2, (0, 127))[None, :], (8, 128))
    bo18 = jnp.broadcast_to(bo1[None, :], (8, _H))
    out = _final(h, batch8, Wo1, bo18, wo2p, bo28)
    return out[:, :1]
